# baseline (speedup 1.0000x reference)
import functools

import jax
import jax.numpy as jnp
from jax.experimental import pallas as pl
from jax.experimental.pallas import tpu as pltpu

LANES = 128
FEATS = 16
PACK = LANES // FEATS
ALIGN = PACK * 8
BLOCK8 = 4096


def _mlp_body(xp_ref, w1_ref, b1_ref, w2_ref, b2_ref, o_ref):
    xp = xp_ref[...]
    h = jnp.dot(xp.astype(jnp.bfloat16), w1_ref[...],
                preferred_element_type=jnp.float32)
    h = jnp.maximum(h + b1_ref[...], 0.0)
    y = jnp.dot(h.astype(jnp.bfloat16), w2_ref[...],
                preferred_element_type=jnp.float32)
    o_ref[...] = (y + b2_ref[...]).astype(o_ref.dtype)


@jax.jit
def _forward(x, w1_blk, b1_blk, w2_blk, b2_blk):
    B, f = x.shape
    Bp = (B + ALIGN - 1) // ALIGN * ALIGN
    if Bp != B:
        x = jnp.pad(x, ((0, Bp - B), (0, 0)))
    n_packed = Bp // PACK
    xp = x.reshape(n_packed, LANES)

    block8 = min(BLOCK8, n_packed)
    grid = (pl.cdiv(n_packed, block8),)

    w1b = w1_blk.astype(jnp.bfloat16)
    w2b = w2_blk.astype(jnp.bfloat16)

    yp = pl.pallas_call(
        _mlp_body,
        out_shape=jax.ShapeDtypeStruct((n_packed, LANES), x.dtype),
        grid=grid,
        in_specs=[
            pl.BlockSpec((block8, LANES), lambda i: (i, 0)),
            pl.BlockSpec((LANES, LANES), lambda i: (0, 0)),
            pl.BlockSpec((1, LANES), lambda i: (0, 0)),
            pl.BlockSpec((LANES, LANES), lambda i: (0, 0)),
            pl.BlockSpec((1, LANES), lambda i: (0, 0)),
        ],
        out_specs=pl.BlockSpec((block8, LANES), lambda i: (i, 0)),
        compiler_params=pltpu.CompilerParams(
            dimension_semantics=("parallel",),
            vmem_limit_bytes=64 << 20,
        ),
    )(xp, w1b, b1_blk, w2b, b2_blk)

    y = yp.reshape(Bp, FEATS)
    return y if Bp == B else y[:B]


def kernel(x, w1_blk, b1_blk, w2_blk, b2_blk):
    return _forward(x, w1_blk, b1_blk, w2_blk, b2_blk)


# speedup vs baseline: 1.0550x; 1.0550x over previous
import functools

import jax
import jax.numpy as jnp
from jax.experimental import pallas as pl
from jax.experimental.pallas import tpu as pltpu

FEATS = 16
BLOCK_ROWS = 16384


def _mlp_body(x_ref, w1_ref, b1_ref, w2_ref, b2_ref, o_ref):
    x = x_ref[...]
    h = jnp.dot(x.astype(jnp.bfloat16), w1_ref[...],
                preferred_element_type=jnp.float32)
    h = jnp.maximum(h + b1_ref[...], 0.0)
    y = jnp.dot(h.astype(jnp.bfloat16), w2_ref[...],
                preferred_element_type=jnp.float32)
    o_ref[...] = (y + b2_ref[...]).astype(o_ref.dtype)


@jax.jit
def _forward(x, w1_blk, b1_blk, w2_blk, b2_blk):
    B, f = x.shape

    w1t = w1_blk[:FEATS, :FEATS].astype(jnp.bfloat16)
    w2t = w2_blk[:FEATS, :FEATS].astype(jnp.bfloat16)
    b1 = b1_blk[:, :FEATS]
    b2 = b2_blk[:, :FEATS]

    block_rows = min(BLOCK_ROWS, B)
    grid = (pl.cdiv(B, block_rows),)

    y = pl.pallas_call(
        _mlp_body,
        out_shape=jax.ShapeDtypeStruct((B, f), x.dtype),
        grid=grid,
        in_specs=[
            pl.BlockSpec((block_rows, FEATS), lambda i: (i, 0)),
            pl.BlockSpec((FEATS, FEATS), lambda i: (0, 0)),
            pl.BlockSpec((1, FEATS), lambda i: (0, 0)),
            pl.BlockSpec((FEATS, FEATS), lambda i: (0, 0)),
            pl.BlockSpec((1, FEATS), lambda i: (0, 0)),
        ],
        out_specs=pl.BlockSpec((block_rows, FEATS), lambda i: (i, 0)),
        compiler_params=pltpu.CompilerParams(
            dimension_semantics=("parallel",),
            vmem_limit_bytes=64 << 20,
        ),
    )(x, w1t, b1, w2t, b2)

    return y


def kernel(x, w1_blk, b1_blk, w2_blk, b2_blk):
    return _forward(x, w1_blk, b1_blk, w2_blk, b2_blk)


# speedup vs baseline: 17.8148x; 16.8867x over previous
import functools

import jax
import jax.numpy as jnp
from jax.experimental import pallas as pl
from jax.experimental.pallas import tpu as pltpu

FEATS = 16
LANES = 128
COL_BLOCK = 65536


def _mlp_t_body(xt_ref, w1_ref, b1_ref, w2_ref, b2_ref, o_ref):
    xt = xt_ref[...]
    h = jnp.dot(w1_ref[...], xt.astype(jnp.bfloat16),
                preferred_element_type=jnp.float32)
    h = jnp.maximum(h + b1_ref[:, 0:1], 0.0)
    y = jnp.dot(w2_ref[...], h.astype(jnp.bfloat16),
                preferred_element_type=jnp.float32)
    o_ref[...] = (y + b2_ref[:, 0:1]).astype(o_ref.dtype)


@jax.jit
def _forward(x, w1_blk, b1_blk, w2_blk, b2_blk):
    B, f = x.shape
    xt = x.T

    w1 = w1_blk[:FEATS, :FEATS].T.astype(jnp.bfloat16)
    w2 = w2_blk[:FEATS, :FEATS].T.astype(jnp.bfloat16)
    b1c = jnp.tile(b1_blk[:1, :FEATS].T, (1, LANES))
    b2c = jnp.tile(b2_blk[:1, :FEATS].T, (1, LANES))

    cb = min(COL_BLOCK, B)
    grid = (pl.cdiv(B, cb),)

    yt = pl.pallas_call(
        _mlp_t_body,
        out_shape=jax.ShapeDtypeStruct((f, B), x.dtype),
        grid=grid,
        in_specs=[
            pl.BlockSpec((FEATS, cb), lambda i: (0, i)),
            pl.BlockSpec((FEATS, FEATS), lambda i: (0, 0)),
            pl.BlockSpec((FEATS, LANES), lambda i: (0, 0)),
            pl.BlockSpec((FEATS, FEATS), lambda i: (0, 0)),
            pl.BlockSpec((FEATS, LANES), lambda i: (0, 0)),
        ],
        out_specs=pl.BlockSpec((FEATS, cb), lambda i: (0, i)),
        compiler_params=pltpu.CompilerParams(
            dimension_semantics=("parallel",),
            vmem_limit_bytes=64 << 20,
        ),
    )(xt, w1, b1c, w2, b2c)

    return yt.T


def kernel(x, w1_blk, b1_blk, w2_blk, b2_blk):
    return _forward(x, w1_blk, b1_blk, w2_blk, b2_blk)


# speedup vs baseline: 18.2903x; 1.0267x over previous
import functools

import jax
import jax.numpy as jnp
from jax.experimental import pallas as pl
from jax.experimental.pallas import tpu as pltpu

FEATS = 16
LANES = 128
COL_BLOCK = 131072


def _mlp_t_body(xt_ref, w1_ref, b1_ref, w2_ref, b2_ref, o_ref):
    xt = xt_ref[...]
    h = jnp.dot(w1_ref[...], xt.astype(jnp.bfloat16),
                preferred_element_type=jnp.float32)
    h = jnp.maximum(h + b1_ref[:, 0:1], 0.0)
    y = jnp.dot(w2_ref[...], h.astype(jnp.bfloat16),
                preferred_element_type=jnp.float32)
    o_ref[...] = (y + b2_ref[:, 0:1]).astype(o_ref.dtype)


@jax.jit
def _forward(x, w1_blk, b1_blk, w2_blk, b2_blk):
    B, f = x.shape
    xt = x.T

    w1 = w1_blk[:FEATS, :FEATS].T.astype(jnp.bfloat16)
    w2 = w2_blk[:FEATS, :FEATS].T.astype(jnp.bfloat16)
    b1c = jnp.tile(b1_blk[:1, :FEATS].T, (1, LANES))
    b2c = jnp.tile(b2_blk[:1, :FEATS].T, (1, LANES))

    cb = min(COL_BLOCK, B)
    grid = (pl.cdiv(B, cb),)

    yt = pl.pallas_call(
        _mlp_t_body,
        out_shape=jax.ShapeDtypeStruct((f, B), x.dtype),
        grid=grid,
        in_specs=[
            pl.BlockSpec((FEATS, cb), lambda i: (0, i)),
            pl.BlockSpec((FEATS, FEATS), lambda i: (0, 0)),
            pl.BlockSpec((FEATS, LANES), lambda i: (0, 0)),
            pl.BlockSpec((FEATS, FEATS), lambda i: (0, 0)),
            pl.BlockSpec((FEATS, LANES), lambda i: (0, 0)),
        ],
        out_specs=pl.BlockSpec((FEATS, cb), lambda i: (0, i)),
        compiler_params=pltpu.CompilerParams(
            dimension_semantics=("parallel",),
            vmem_limit_bytes=64 << 20,
        ),
    )(xt, w1, b1c, w2, b2c)

    return yt.T


def kernel(x, w1_blk, b1_blk, w2_blk, b2_blk):
    return _forward(x, w1_blk, b1_blk, w2_blk, b2_blk)
